# baseline (speedup 1.0000x reference)
import functools

import jax
import jax.numpy as jnp
from jax import lax
from jax.experimental import pallas as pl
from jax.experimental.pallas import tpu as pltpu
from jax.experimental.pallas import tpu_sc as plsc

N = 10000
E = 320000
D = 128

NC = 2
NS = 16
NW = NC * NS
ET = E // NW
C = 128
GR = 80
GG = 2
GPR = GR // GG
NPAD = 10112
RPT = NPAD // NS
CG = 80
CP = jax.ShapeDtypeStruct


def _sc_agg(x, src2, dst2):
  mesh = plsc.VectorSubcoreMesh(core_axis_name="c", subcore_axis_name="s")

  @functools.partial(
      pl.kernel,
      out_type=CP((NC, NPAD, D), jnp.float32),
      mesh=mesh,
      scratch_types=[
          pltpu.VMEM((GPR, C), jnp.int32),
          pltpu.VMEM((GPR, C), jnp.int32),
          pltpu.VMEM((C, D), jnp.float32),
          pltpu.VMEM((C, D), jnp.float32),
          pltpu.VMEM_SHARED((NPAD, D), jnp.float32),
          pltpu.SemaphoreType.DMA,
          pltpu.SemaphoreType.DMA,
      ],
      compiler_params=pltpu.CompilerParams(needs_layout_passes=False),
  )
  def k(x_hbm, src_hbm, dst_hbm, agg_out, src_v, dst_v, rows0, rows1, agg_sh,
        sem0, sem1):
    cid = lax.axis_index("c")
    sid = lax.axis_index("s")
    wid = cid * NS + sid
    base = wid * GR

    zeros16 = jnp.zeros((16,), jnp.float32)

    def zrow(i, carry):
      def zcol(c, carry2):
        rows0[i, pl.ds(c * 16, 16)] = zeros16
        return carry2
      return lax.fori_loop(0, D // 16, zcol, carry)

    lax.fori_loop(0, 8, zrow, 0)

    def zacc(i, carry):
      pltpu.sync_copy(rows0.at[pl.ds(0, 8)],
                      agg_sh.at[pl.ds(sid * RPT + i * 8, 8)])
      return carry

    lax.fori_loop(0, RPT // 8, zacc, 0)
    plsc.subcore_barrier()

    for g, gn in ((0, GPR), (1, GPR - 1)):
      pltpu.sync_copy(src_hbm.at[pl.ds(base + g * GPR, GPR)], src_v)
      pltpu.sync_copy(dst_hbm.at[pl.ds(base + g * GPR, GPR)], dst_v)
      pltpu.async_copy(x_hbm.at[src_v.at[0]], rows0, sem0)

      def body(j, carry):
        @pl.when(j + 1 < gn)
        def _pref():
          @pl.when(lax.rem(j, 2) == 0)
          def _():
            pltpu.async_copy(x_hbm.at[src_v.at[j + 1]], rows1, sem1)
          @pl.when(lax.rem(j, 2) == 1)
          def _():
            pltpu.async_copy(x_hbm.at[src_v.at[j + 1]], rows0, sem0)

        @pl.when(lax.rem(j, 2) == 0)
        def _even():
          pltpu.make_async_copy(x_hbm.at[src_v.at[j]], rows0, sem0).wait()
          pltpu.sync_copy(rows0, agg_sh.at[dst_v.at[j]], add=True)

        @pl.when(lax.rem(j, 2) == 1)
        def _odd():
          pltpu.make_async_copy(x_hbm.at[src_v.at[j]], rows1, sem1).wait()
          pltpu.sync_copy(rows1, agg_sh.at[dst_v.at[j]], add=True)

        return carry

      lax.fori_loop(0, gn, body, 0)
    plsc.subcore_barrier()

    pltpu.sync_copy(agg_sh.at[pl.ds(sid * RPT, RPT)],
                    agg_out.at[cid, pl.ds(sid * RPT, RPT)])

  return k(x, src2, dst2)


def _sc_cnt(dst2):
  mesh = plsc.VectorSubcoreMesh(core_axis_name="c", subcore_axis_name="s")

  @functools.partial(
      pl.kernel,
      out_type=CP((NC, CG, D), jnp.float32),
      mesh=mesh,
      scratch_types=[
          pltpu.VMEM((GR, C), jnp.int32),
          pltpu.VMEM((CG, D), jnp.float32),
          pltpu.VMEM((CG,), jnp.int32),
          pltpu.VMEM_SHARED((CG, D), jnp.float32),
      ],
      compiler_params=pltpu.CompilerParams(needs_layout_passes=False),
  )
  def k(dst_hbm, cnt_out, dst_v, cnt_v, iota_v, cnt_sh):
    cid = lax.axis_index("c")
    sid = lax.axis_index("s")
    wid = cid * NS + sid

    pltpu.sync_copy(dst_hbm.at[pl.ds(wid * GR, GR)], dst_v)

    zeros16 = jnp.zeros((16,), jnp.float32)
    lanes16 = lax.iota(jnp.int32, 16)

    def crow(i, carry):
      def ccol(c, carry2):
        cnt_v[i, pl.ds(c * 16, 16)] = zeros16
        return carry2
      return lax.fori_loop(0, D // 16, ccol, carry)

    lax.fori_loop(0, CG, crow, 0)

    def irow(i, carry):
      iota_v[pl.ds(i * 16, 16)] = lanes16 + i * 16
      return carry

    lax.fori_loop(0, CG // 16, irow, 0)

    @pl.when(sid < CG // 8)
    def _zcnt():
      pltpu.sync_copy(cnt_v.at[pl.ds(0, 8)], cnt_sh.at[pl.ds(sid * 8, 8)])

    plsc.subcore_barrier()

    ones16 = jnp.full((16,), 1.0, jnp.float32)

    def body(j, carry):
      for kk in range(C // 16):
        idx = dst_v[j, pl.ds(kk * 16, 16)]
        plsc.addupdate_scatter(
            cnt_v, [lax.shift_right_logical(idx, 7),
                    jnp.bitwise_and(idx, 127)], ones16)
      return carry

    lax.fori_loop(0, GR, body, 0)

    pltpu.sync_copy(cnt_v, cnt_sh.at[iota_v], add=True)
    plsc.subcore_barrier()

    @pl.when(sid < CG // 8)
    def _ccnt():
      pltpu.sync_copy(cnt_sh.at[pl.ds(sid * 8, 8)],
                      cnt_out.at[cid, pl.ds(sid * 8, 8)])

  return k(dst2)


def _tc_tail(agg2, cnt2, x, W_l, b_l, W_r, gamma, beta):
  def body(agg_ref, cnt_ref, x_ref, wl_ref, bl_ref, wr_ref, g_ref, b_ref,
           o_ref):
    a = (agg_ref[0] + agg_ref[1])[:N]
    cnt = jnp.maximum(cnt_ref[0] + cnt_ref[1], 1.0)
    mean = a / cnt
    xv = x_ref[...]
    pre = (
        lax.dot_general(mean, wl_ref[...], (((1,), (1,)), ((), ())),
                        preferred_element_type=jnp.float32)
        + bl_ref[...]
        + lax.dot_general(xv, wr_ref[...], (((1,), (1,)), ((), ())),
                          preferred_element_type=jnp.float32)
    )
    mu = jnp.mean(pre, axis=0, keepdims=True)
    var = jnp.mean((pre - mu) ** 2, axis=0, keepdims=True)
    y = (pre - mu) * lax.rsqrt(var + 1e-5) * g_ref[...] + b_ref[...]
    y = jnp.where(y >= 0, y, 0.01 * y)
    o_ref[...] = y + xv

  return pl.pallas_call(
      body,
      out_shape=CP((N, D), jnp.float32),
  )(agg2, cnt2, x, W_l.reshape(D, D), b_l.reshape(1, D), W_r.reshape(D, D),
    gamma.reshape(1, D), beta.reshape(1, D))


def kernel(x, edge_index, W_l, b_l, W_r, gamma, beta):
  npd = GR * C - ET
  sdst = N + jnp.arange(npd, dtype=jnp.int32) % (NPAD - N)
  src2 = jnp.concatenate(
      [edge_index[0].reshape(NW, ET),
       jnp.zeros((NW, npd), jnp.int32)], axis=1).reshape(NW * GR, C)
  dst2 = jnp.concatenate(
      [edge_index[1].reshape(NW, ET),
       jnp.broadcast_to(sdst, (NW, npd))], axis=1).reshape(NW * GR, C)
  agg2 = _sc_agg(x, src2, dst2)
  cntg = _sc_cnt(dst2)
  cnt2 = cntg.reshape(NC, CG * D, 1)[:, :N]
  return _tc_tail(agg2, cnt2, x, W_l, b_l, W_r, gamma, beta)


# speedup vs baseline: 1.7620x; 1.7620x over previous
import functools

import jax
import jax.numpy as jnp
from jax import lax
from jax.experimental import pallas as pl
from jax.experimental.pallas import tpu as pltpu
from jax.experimental.pallas import tpu_sc as plsc

N = 10000
E = 320000
D = 128

NC = 2
NS = 16
NW = NC * NS
ET = E // NW
C = 128
GR = 80
GG = 2
GPR = GR // GG
NPAD = 10112
RPT = NPAD // NS
CG = 80
CP = jax.ShapeDtypeStruct


def _sc_agg(x, src2, dst2):
  mesh = plsc.VectorSubcoreMesh(core_axis_name="c", subcore_axis_name="s")

  @functools.partial(
      pl.kernel,
      out_type=CP((NC, NPAD, D), jnp.float32),
      mesh=mesh,
      scratch_types=[
          pltpu.VMEM((GPR, C), jnp.int32),
          pltpu.VMEM((GPR, C), jnp.int32),
          pltpu.VMEM((C, D), jnp.float32),
          pltpu.VMEM((C, D), jnp.float32),
          pltpu.VMEM_SHARED((NPAD, D), jnp.float32),
          pltpu.SemaphoreType.DMA,
          pltpu.SemaphoreType.DMA,
      ],
      compiler_params=pltpu.CompilerParams(needs_layout_passes=False),
  )
  def k(x_hbm, src_hbm, dst_hbm, agg_out, src_v, dst_v, rows0, rows1, agg_sh,
        sem0, sem1):
    cid = lax.axis_index("c")
    sid = lax.axis_index("s")
    wid = cid * NS + sid
    base = wid * GR

    zeros16 = jnp.zeros((16,), jnp.float32)

    def zrow(i, carry):
      def zcol(c, carry2):
        rows0[i, pl.ds(c * 16, 16)] = zeros16
        return carry2
      return lax.fori_loop(0, D // 16, zcol, carry)

    lax.fori_loop(0, 8, zrow, 0)

    def zacc(i, carry):
      pltpu.sync_copy(rows0.at[pl.ds(0, 8)],
                      agg_sh.at[pl.ds(sid * RPT + i * 8, 8)])
      return carry

    lax.fori_loop(0, RPT // 8, zacc, 0)
    plsc.subcore_barrier()

    for g, gn in ((0, GPR), (1, GPR - 1)):
      pltpu.sync_copy(src_hbm.at[pl.ds(base + g * GPR, GPR)], src_v)
      pltpu.sync_copy(dst_hbm.at[pl.ds(base + g * GPR, GPR)], dst_v)
      pltpu.async_copy(x_hbm.at[src_v.at[0]], rows0, sem0)

      def body(j, carry):
        @pl.when(j + 1 < gn)
        def _pref():
          @pl.when(lax.rem(j, 2) == 0)
          def _():
            pltpu.async_copy(x_hbm.at[src_v.at[j + 1]], rows1, sem1)
          @pl.when(lax.rem(j, 2) == 1)
          def _():
            pltpu.async_copy(x_hbm.at[src_v.at[j + 1]], rows0, sem0)

        @pl.when(lax.rem(j, 2) == 0)
        def _even():
          pltpu.make_async_copy(x_hbm.at[src_v.at[j]], rows0, sem0).wait()
          pltpu.sync_copy(rows0, agg_sh.at[dst_v.at[j]], add=True)

        @pl.when(lax.rem(j, 2) == 1)
        def _odd():
          pltpu.make_async_copy(x_hbm.at[src_v.at[j]], rows1, sem1).wait()
          pltpu.sync_copy(rows1, agg_sh.at[dst_v.at[j]], add=True)

        return carry

      lax.fori_loop(0, gn, body, 0)
    plsc.subcore_barrier()

    pltpu.sync_copy(agg_sh.at[pl.ds(sid * RPT, RPT)],
                    agg_out.at[cid, pl.ds(sid * RPT, RPT)])

  return k(x, src2, dst2)


def _sc_cnt(dst2):
  mesh = plsc.VectorSubcoreMesh(core_axis_name="c", subcore_axis_name="s")

  @functools.partial(
      pl.kernel,
      out_type=CP((NC, CG, D), jnp.float32),
      mesh=mesh,
      scratch_types=[
          pltpu.VMEM((GR, C), jnp.int32),
          pltpu.VMEM((CG, D), jnp.float32),
          pltpu.VMEM((CG,), jnp.int32),
          pltpu.VMEM_SHARED((CG, D), jnp.float32),
      ],
      compiler_params=pltpu.CompilerParams(needs_layout_passes=False),
  )
  def k(dst_hbm, cnt_out, dst_v, cnt_v, iota_v, cnt_sh):
    cid = lax.axis_index("c")
    sid = lax.axis_index("s")
    wid = cid * NS + sid

    pltpu.sync_copy(dst_hbm.at[pl.ds(wid * GR, GR)], dst_v)

    zeros16 = jnp.zeros((16,), jnp.float32)
    lanes16 = lax.iota(jnp.int32, 16)

    def crow(i, carry):
      def ccol(c, carry2):
        cnt_v[i, pl.ds(c * 16, 16)] = zeros16
        return carry2
      return lax.fori_loop(0, D // 16, ccol, carry)

    lax.fori_loop(0, CG, crow, 0)

    def irow(i, carry):
      iota_v[pl.ds(i * 16, 16)] = lanes16 + i * 16
      return carry

    lax.fori_loop(0, CG // 16, irow, 0)

    @pl.when(sid < CG // 8)
    def _zcnt():
      pltpu.sync_copy(cnt_v.at[pl.ds(0, 8)], cnt_sh.at[pl.ds(sid * 8, 8)])

    plsc.subcore_barrier()

    ones16 = jnp.full((16,), 1.0, jnp.float32)

    def body(j, carry):
      for kk in range(C // 16):
        idx = dst_v[j, pl.ds(kk * 16, 16)]
        plsc.addupdate_scatter(
            cnt_v, [lax.shift_right_logical(idx, 7),
                    jnp.bitwise_and(idx, 127)], ones16)
      return carry

    lax.fori_loop(0, GR, body, 0)

    pltpu.sync_copy(cnt_v, cnt_sh.at[iota_v], add=True)
    plsc.subcore_barrier()

    @pl.when(sid < CG // 8)
    def _ccnt():
      pltpu.sync_copy(cnt_sh.at[pl.ds(sid * 8, 8)],
                      cnt_out.at[cid, pl.ds(sid * 8, 8)])

  return k(dst2)


def _tc_tail(agg2, cnt2, x, W_l, b_l, W_r, gamma, beta):
  def body(agg_ref, cnt_ref, x_ref, wl_ref, bl_ref, wr_ref, g_ref, b_ref,
           o_ref):
    a = (agg_ref[0] + agg_ref[1])[:N]
    cnt = jnp.maximum(cnt_ref[0] + cnt_ref[1], 1.0)
    mean = a / cnt
    xv = x_ref[...]
    pre = (
        lax.dot_general(mean, wl_ref[...], (((1,), (1,)), ((), ())),
                        preferred_element_type=jnp.float32)
        + bl_ref[...]
        + lax.dot_general(xv, wr_ref[...], (((1,), (1,)), ((), ())),
                          preferred_element_type=jnp.float32)
    )
    mu = jnp.mean(pre, axis=0, keepdims=True)
    var = jnp.mean((pre - mu) ** 2, axis=0, keepdims=True)
    y = (pre - mu) * lax.rsqrt(var + 1e-5) * g_ref[...] + b_ref[...]
    y = jnp.where(y >= 0, y, 0.01 * y)
    o_ref[...] = y + xv

  return pl.pallas_call(
      body,
      out_shape=CP((N, D), jnp.float32),
  )(agg2, cnt2, x, W_l.reshape(D, D), b_l.reshape(1, D), W_r.reshape(D, D),
    gamma.reshape(1, D), beta.reshape(1, D))


def kernel(x, edge_index, W_l, b_l, W_r, gamma, beta):
  npd = GR * C - ET
  sdst = N + jnp.arange(npd, dtype=jnp.int32) % (NPAD - N)
  ssrc = jnp.arange(npd, dtype=jnp.int32) * 89 % N
  src2 = jnp.concatenate(
      [edge_index[0].reshape(NW, ET),
       jnp.broadcast_to(ssrc, (NW, npd))], axis=1).reshape(NW * GR, C)
  dst2 = jnp.concatenate(
      [edge_index[1].reshape(NW, ET),
       jnp.broadcast_to(sdst, (NW, npd))], axis=1).reshape(NW * GR, C)
  agg2 = _sc_agg(x, src2, dst2)
  cntg = _sc_cnt(dst2)
  cnt2 = cntg.reshape(NC, CG * D, 1)[:, :N]
  return _tc_tail(agg2, cnt2, x, W_l, b_l, W_r, gamma, beta)
